# Initial kernel scaffold; baseline (speedup 1.0000x reference)
import jax
import jax.numpy as jnp
from jax.experimental import pallas as pl


def kernel(x, att_node, edge_index, rel, att_edge, W_node, b_node, W_rel, b_rel, W_apply, b_apply):
    raise NotImplementedError("write your pallas kernel here")


# speedup vs baseline: 6.6280x; 6.6280x over previous
import functools

import jax
import jax.numpy as jnp
from jax import lax
from jax.experimental import pallas as pl
from jax.experimental.pallas import tpu as pltpu
from jax.experimental.pallas import tpu_sc as plsc

N = 10000
E = 320000
D = 128
DH = D // 2
DR = 16

NC = 2
NS = 16
NP = 10240
ROWS_PER_TILE = NP // NS
B = 80
E_PER_TILE = E // NS
P2_CHUNKS = E_PER_TILE // B
E_PER_W = E // (NC * NS)
QD_CHUNKS = E_PER_W // B

_NODE_BLK = 1000
_EDGE_BLK = 4000


def _node_pre_body(x_ref, att_ref, wnt_ref, bn_ref, h_ref, g2a_ref, g2b_ref):
    h = jnp.dot(x_ref[...], wnt_ref[...], preferred_element_type=jnp.float32)
    h = h + bn_ref[...]
    h_ref[...] = h
    g2 = att_ref[...] * h
    g2a_ref[...] = g2[:, :DH]
    g2b_ref[...] = g2[:, DH:]


def _node_pre(x, att_node, wn_t, b_node):
    grid = (N // _NODE_BLK,)
    return pl.pallas_call(
        _node_pre_body,
        grid=grid,
        in_specs=[
            pl.BlockSpec((_NODE_BLK, D), lambda i: (i, 0)),
            pl.BlockSpec((_NODE_BLK, 1), lambda i: (i, 0)),
            pl.BlockSpec((D, D), lambda i: (0, 0)),
            pl.BlockSpec((1, D), lambda i: (0, 0)),
        ],
        out_specs=[
            pl.BlockSpec((_NODE_BLK, D), lambda i: (i, 0)),
            pl.BlockSpec((_NODE_BLK, DH), lambda i: (i, 0)),
            pl.BlockSpec((_NODE_BLK, DH), lambda i: (i, 0)),
        ],
        out_shape=[
            jax.ShapeDtypeStruct((N, D), jnp.float32),
            jax.ShapeDtypeStruct((N, DH), jnp.float32),
            jax.ShapeDtypeStruct((N, DH), jnp.float32),
        ],
    )(x, att_node, wn_t, b_node)


def _edge_pre_body(rel_ref, ae_ref, wrt_ref, br_ref, out_ref):
    r2 = jnp.dot(rel_ref[...], wrt_ref[...], preferred_element_type=jnp.float32)
    r2 = ae_ref[...] * (r2 + br_ref[...])
    pad = (lax.broadcasted_iota(jnp.int32, (_EDGE_BLK, DR), 1) == 0)
    out_ref[...] = jnp.concatenate([r2, pad.astype(jnp.float32)], axis=1)


def _edge_pre(rel, att_edge, wr_t, b_rel):
    grid = (E // _EDGE_BLK,)
    return pl.pallas_call(
        _edge_pre_body,
        grid=grid,
        in_specs=[
            pl.BlockSpec((_EDGE_BLK, DR), lambda i: (i, 0)),
            pl.BlockSpec((_EDGE_BLK, 1), lambda i: (i, 0)),
            pl.BlockSpec((DR, DR), lambda i: (0, 0)),
            pl.BlockSpec((1, DR), lambda i: (0, 0)),
        ],
        out_specs=pl.BlockSpec((_EDGE_BLK, 2 * DR), lambda i: (i, 0)),
        out_shape=jax.ShapeDtypeStruct((E, 2 * DR), jnp.float32),
    )(rel, att_edge, wr_t, b_rel)


_sc_mesh = plsc.VectorSubcoreMesh(core_axis_name="c", subcore_axis_name="s")


@functools.partial(
    pl.kernel,
    out_type=(
        jax.ShapeDtypeStruct((NC, NP, DH), jnp.float32),
        jax.ShapeDtypeStruct((NC, NP, 2 * DR), jnp.float32),
    ),
    mesh=_sc_mesh,
    compiler_params=pltpu.CompilerParams(use_tc_tiling_on_sc=False),
    scratch_types=[
        pltpu.VMEM((B,), jnp.int32),
        pltpu.VMEM((B,), jnp.int32),
        pltpu.VMEM((B, DH), jnp.float32),
        pltpu.VMEM((B, 2 * DR), jnp.float32),
        pltpu.VMEM((160, DH), jnp.float32),
        pltpu.VMEM((ROWS_PER_TILE, 2 * DR), jnp.float32),
        pltpu.VMEM_SHARED((NP, DH), jnp.float32),
        pltpu.VMEM_SHARED((NP, 2 * DR), jnp.float32),
        pltpu.SemaphoreType.DMA,
    ],
)
def _sc_edge_agg(g2a_hbm, g2b_hbm, src_hbm, dst_hbm, r2_hbm, p2_out, qd_out,
                 src_v, dst_v, rows_v, r2_v, zb_w, zb_n, p2_sh, qd_sh, sem):
    c = lax.axis_index("c")
    s = lax.axis_index("s")
    wid = c * NS + s
    zero16 = jnp.zeros((16,), jnp.float32)

    def zrow_w(i, carry):
        for j in range(DH // 16):
            zb_w[i, pl.ds(j * 16, 16)] = zero16
        return carry

    lax.fori_loop(0, 160, zrow_w, 0)

    def zrow_n(i, carry):
        for j in range(2 * DR // 16):
            zb_n[i, pl.ds(j * 16, 16)] = zero16
        return carry

    lax.fori_loop(0, ROWS_PER_TILE, zrow_n, 0)

    base_r = s * ROWS_PER_TILE
    for k in range(ROWS_PER_TILE // 160):
        pltpu.sync_copy(zb_w, p2_sh.at[pl.ds(base_r + k * 160, 160)])
    pltpu.sync_copy(zb_n, qd_sh.at[pl.ds(base_r, ROWS_PER_TILE)])
    plsc.subcore_barrier()

    p2_e0 = s * E_PER_TILE

    def p2_chunk(i, carry, g2_hbm):
        base = p2_e0 + i * B
        pltpu.sync_copy(src_hbm.at[pl.ds(base, B)], src_v)
        pltpu.sync_copy(dst_hbm.at[pl.ds(base, B)], dst_v)
        pltpu.async_copy(g2_hbm.at[src_v], rows_v, sem).wait()
        pltpu.sync_copy(rows_v, p2_sh.at[dst_v], add=True)
        return carry

    @pl.when(c == 0)
    def _():
        lax.fori_loop(0, P2_CHUNKS, functools.partial(p2_chunk, g2_hbm=g2a_hbm), 0)

    @pl.when(c == 1)
    def _():
        lax.fori_loop(0, P2_CHUNKS, functools.partial(p2_chunk, g2_hbm=g2b_hbm), 0)

    qd_e0 = wid * E_PER_W

    def qd_chunk(i, carry):
        base = qd_e0 + i * B
        pltpu.sync_copy(dst_hbm.at[pl.ds(base, B)], dst_v)
        pltpu.sync_copy(r2_hbm.at[pl.ds(base, B)], r2_v)
        pltpu.sync_copy(r2_v, qd_sh.at[dst_v], add=True)
        return carry

    lax.fori_loop(0, QD_CHUNKS, qd_chunk, 0)
    plsc.subcore_barrier()

    pltpu.sync_copy(p2_sh.at[pl.ds(base_r, ROWS_PER_TILE)],
                    p2_out.at[c, pl.ds(base_r, ROWS_PER_TILE)])
    pltpu.sync_copy(qd_sh.at[pl.ds(base_r, ROWS_PER_TILE)],
                    qd_out.at[c, pl.ds(base_r, ROWS_PER_TILE)])


def _post_body(h_ref, g2a_ref, g2b_ref, p2p_ref, qdp_ref, waht_ref, wa1t_ref,
               wa2t_ref, ba_ref, out_ref):
    p2 = jnp.concatenate([p2p_ref[0], p2p_ref[1]], axis=1)
    qd = qdp_ref[0] + qdp_ref[1]
    deg = qd[:, DR:DR + 1]
    qr = qd[:, 0:DR]
    g2 = jnp.concatenate([g2a_ref[...], g2b_ref[...]], axis=1)
    hs1 = p2 + deg * g2
    acc = jnp.dot(h_ref[...], waht_ref[...], preferred_element_type=jnp.float32)
    acc = acc + jnp.dot(hs1, wa1t_ref[...], preferred_element_type=jnp.float32)
    acc = acc + jnp.dot(qr, wa2t_ref[...], preferred_element_type=jnp.float32)
    out_ref[...] = jnp.maximum(acc + ba_ref[...], 0.0)


def _post(h, g2a, g2b, p2p, qdp, wah_t, wa1_t, wa2_t, b_apply):
    grid = (N // _NODE_BLK,)
    return pl.pallas_call(
        _post_body,
        grid=grid,
        in_specs=[
            pl.BlockSpec((_NODE_BLK, D), lambda i: (i, 0)),
            pl.BlockSpec((_NODE_BLK, DH), lambda i: (i, 0)),
            pl.BlockSpec((_NODE_BLK, DH), lambda i: (i, 0)),
            pl.BlockSpec((NC, _NODE_BLK, DH), lambda i: (0, i, 0)),
            pl.BlockSpec((NC, _NODE_BLK, 2 * DR), lambda i: (0, i, 0)),
            pl.BlockSpec((D, D), lambda i: (0, 0)),
            pl.BlockSpec((D, D), lambda i: (0, 0)),
            pl.BlockSpec((DR, D), lambda i: (0, 0)),
            pl.BlockSpec((1, D), lambda i: (0, 0)),
        ],
        out_specs=pl.BlockSpec((_NODE_BLK, D), lambda i: (i, 0)),
        out_shape=jax.ShapeDtypeStruct((N, D), jnp.float32),
    )(h, g2a, g2b, p2p, qdp, wah_t, wa1_t, wa2_t, b_apply)


def kernel(x, att_node, edge_index, rel, att_edge, W_node, b_node, W_rel,
           b_rel, W_apply, b_apply):
    src = edge_index[0].astype(jnp.int32)
    dst = edge_index[1].astype(jnp.int32)
    h, g2a, g2b = _node_pre(x, att_node, W_node.T, b_node.reshape(1, D))
    r2aug = _edge_pre(rel, att_edge, W_rel.T, b_rel.reshape(1, DR))
    p2p, qdp = _sc_edge_agg(g2a, g2b, src, dst, r2aug)
    return _post(h, g2a, g2b, p2p, qdp,
                 W_apply[:, :D].T, W_apply[:, D:2 * D].T, W_apply[:, 2 * D:].T,
                 b_apply.reshape(1, D))


# speedup vs baseline: 11.0140x; 1.6617x over previous
import functools

import jax
import jax.numpy as jnp
from jax import lax
from jax.experimental import pallas as pl
from jax.experimental.pallas import tpu as pltpu
from jax.experimental.pallas import tpu_sc as plsc

N = 10000
E = 320000
D = 128
DH = D // 2
DR = 16

NC = 2
NS = 16
NP = 10240
ROWS_PER_TILE = NP // NS
B = 128
E_PER_TILE = E // NS
NB = E_PER_TILE // B
BT = E_PER_TILE - NB * B
QD_NB = NB // 2

_NODE_BLK = 1000
_EDGE_BLK = 4000


def _node_pre_body(x_ref, att_ref, wnt_ref, bn_ref, h_ref, g2a_ref, g2b_ref):
    h = jnp.dot(x_ref[...], wnt_ref[...], preferred_element_type=jnp.float32)
    h = h + bn_ref[...]
    h_ref[...] = h
    g2 = att_ref[...] * h
    g2a_ref[...] = g2[:, :DH]
    g2b_ref[...] = g2[:, DH:]


def _node_pre(x, att_node, wn_t, b_node):
    grid = (N // _NODE_BLK,)
    return pl.pallas_call(
        _node_pre_body,
        grid=grid,
        in_specs=[
            pl.BlockSpec((_NODE_BLK, D), lambda i: (i, 0)),
            pl.BlockSpec((_NODE_BLK, 1), lambda i: (i, 0)),
            pl.BlockSpec((D, D), lambda i: (0, 0)),
            pl.BlockSpec((1, D), lambda i: (0, 0)),
        ],
        out_specs=[
            pl.BlockSpec((_NODE_BLK, D), lambda i: (i, 0)),
            pl.BlockSpec((_NODE_BLK, DH), lambda i: (i, 0)),
            pl.BlockSpec((_NODE_BLK, DH), lambda i: (i, 0)),
        ],
        out_shape=[
            jax.ShapeDtypeStruct((N, D), jnp.float32),
            jax.ShapeDtypeStruct((N, DH), jnp.float32),
            jax.ShapeDtypeStruct((N, DH), jnp.float32),
        ],
    )(x, att_node, wn_t, b_node)


def _edge_pre_body(rel_ref, ae_ref, wrt_ref, br_ref, out_ref):
    r2 = jnp.dot(rel_ref[...], wrt_ref[...], preferred_element_type=jnp.float32)
    r2 = ae_ref[...] * (r2 + br_ref[...])
    pad = (lax.broadcasted_iota(jnp.int32, (_EDGE_BLK, DR), 1) == 0)
    out_ref[...] = jnp.concatenate([r2, pad.astype(jnp.float32)], axis=1)


def _edge_pre(rel, att_edge, wr_t, b_rel):
    grid = (E // _EDGE_BLK,)
    return pl.pallas_call(
        _edge_pre_body,
        grid=grid,
        in_specs=[
            pl.BlockSpec((_EDGE_BLK, DR), lambda i: (i, 0)),
            pl.BlockSpec((_EDGE_BLK, 1), lambda i: (i, 0)),
            pl.BlockSpec((DR, DR), lambda i: (0, 0)),
            pl.BlockSpec((1, DR), lambda i: (0, 0)),
        ],
        out_specs=pl.BlockSpec((_EDGE_BLK, 2 * DR), lambda i: (i, 0)),
        out_shape=jax.ShapeDtypeStruct((E, 2 * DR), jnp.float32),
    )(rel, att_edge, wr_t, b_rel)


_sc_mesh = plsc.VectorSubcoreMesh(core_axis_name="c", subcore_axis_name="s")


@functools.partial(
    pl.kernel,
    out_type=(
        jax.ShapeDtypeStruct((NC, NP, DH), jnp.float32),
        jax.ShapeDtypeStruct((NC, NP, 2 * DR), jnp.float32),
    ),
    mesh=_sc_mesh,
    compiler_params=pltpu.CompilerParams(use_tc_tiling_on_sc=False),
    scratch_types=[
        pltpu.VMEM((2, B), jnp.int32),
        pltpu.VMEM((2, B), jnp.int32),
        pltpu.VMEM((B, DH), jnp.float32),
        pltpu.VMEM((B, DH), jnp.float32),
        pltpu.VMEM((B, 2 * DR), jnp.float32),
        pltpu.VMEM((B, 2 * DR), jnp.float32),
        pltpu.VMEM((2, BT), jnp.int32),
        pltpu.VMEM((BT, DH), jnp.float32),
        pltpu.VMEM((BT, 2 * DR), jnp.float32),
        pltpu.VMEM((160, DH), jnp.float32),
        pltpu.VMEM((ROWS_PER_TILE, 2 * DR), jnp.float32),
        pltpu.VMEM_SHARED((NP, DH), jnp.float32),
        pltpu.VMEM_SHARED((NP, 2 * DR), jnp.float32),
        pltpu.SemaphoreType.DMA,
        pltpu.SemaphoreType.DMA,
        pltpu.SemaphoreType.DMA,
        pltpu.SemaphoreType.DMA,
    ],
)
def _sc_edge_agg(g2a_hbm, g2b_hbm, ei_hbm, r2_hbm, p2_out, qd_out,
                 idx0, idx1, rows0, rows1, r20, r21, idx_t, rows_t, r2_t,
                 zb_w, zb_n, p2_sh, qd_sh, semi0, semi1, semg0, semg1):
    c = lax.axis_index("c")
    s = lax.axis_index("s")
    zero16 = jnp.zeros((16,), jnp.float32)

    def zrow_w(i, carry):
        for j in range(DH // 16):
            zb_w[i, pl.ds(j * 16, 16)] = zero16
        return carry

    lax.fori_loop(0, 160, zrow_w, 0)

    def zrow_n(i, carry):
        for j in range(2 * DR // 16):
            zb_n[i, pl.ds(j * 16, 16)] = zero16
        return carry

    lax.fori_loop(0, ROWS_PER_TILE, zrow_n, 0)

    base_r = s * ROWS_PER_TILE
    for k in range(ROWS_PER_TILE // 160):
        pltpu.sync_copy(zb_w, p2_sh.at[pl.ds(base_r + k * 160, 160)])
    pltpu.sync_copy(zb_n, qd_sh.at[pl.ds(base_r, ROWS_PER_TILE)])
    plsc.subcore_barrier()

    tbase = s * E_PER_TILE

    def iload(b, slot, sem):
        return pltpu.async_copy(ei_hbm.at[:, pl.ds(tbase + b * B, B)], slot, sem)

    def iwait(slot, sem):
        pltpu.make_async_copy(ei_hbm.at[:, pl.ds(tbase, B)], slot, sem).wait()

    def gstart(g2_hbm, slot_i, slot_r, sem):
        return pltpu.async_copy(g2_hbm.at[slot_i.at[0]], slot_r, sem)

    def gwait(g2_hbm, slot_r, sem):
        pltpu.make_async_copy(g2_hbm.at[pl.ds(0, B)], slot_r, sem).wait()

    def rload(b, slot, sem):
        return pltpu.async_copy(r2_hbm.at[pl.ds(tbase + b * B, B)], slot, sem)

    def rwait(slot, sem):
        pltpu.make_async_copy(r2_hbm.at[pl.ds(tbase, B)], slot, sem).wait()

    def p2_pipeline(g2_hbm):
        iload(0, idx0, semi0)
        iload(1, idx1, semi1)
        iwait(idx0, semi0)
        gstart(g2_hbm, idx0, rows0, semg0)

        def outer(bb, carry):
            b0 = bb * 2
            iwait(idx1, semi1)
            gstart(g2_hbm, idx1, rows1, semg1)
            gwait(g2_hbm, rows0, semg0)
            pltpu.sync_copy(rows0, p2_sh.at[idx0.at[1]], add=True)
            iload(b0 + 2, idx0, semi0)
            iwait(idx0, semi0)
            gstart(g2_hbm, idx0, rows0, semg0)
            gwait(g2_hbm, rows1, semg1)
            pltpu.sync_copy(rows1, p2_sh.at[idx1.at[1]], add=True)
            iload(b0 + 3, idx1, semi1)
            return carry

        lax.fori_loop(0, NB // 2 - 1, outer, 0)
        iwait(idx1, semi1)
        gstart(g2_hbm, idx1, rows1, semg1)
        gwait(g2_hbm, rows0, semg0)
        pltpu.sync_copy(rows0, p2_sh.at[idx0.at[1]], add=True)
        gwait(g2_hbm, rows1, semg1)
        pltpu.sync_copy(rows1, p2_sh.at[idx1.at[1]], add=True)
        pltpu.sync_copy(ei_hbm.at[:, pl.ds(tbase + NB * B, BT)], idx_t)
        pltpu.async_copy(g2_hbm.at[idx_t.at[0]], rows_t, semg0).wait()
        pltpu.sync_copy(rows_t, p2_sh.at[idx_t.at[1]], add=True)

    def qd_pipeline(b_lo, with_tail):
        iload(b_lo, idx0, semi0)
        rload(b_lo, r20, semg0)
        iload(b_lo + 1, idx1, semi1)
        rload(b_lo + 1, r21, semg1)

        def outer(bb, carry):
            b0 = b_lo + bb * 2
            iwait(idx0, semi0)
            rwait(r20, semg0)
            pltpu.sync_copy(r20, qd_sh.at[idx0.at[1]], add=True)
            iload(b0 + 2, idx0, semi0)
            rload(b0 + 2, r20, semg0)
            iwait(idx1, semi1)
            rwait(r21, semg1)
            pltpu.sync_copy(r21, qd_sh.at[idx1.at[1]], add=True)
            iload(b0 + 3, idx1, semi1)
            rload(b0 + 3, r21, semg1)
            return carry

        lax.fori_loop(0, QD_NB // 2 - 1, outer, 0)
        iwait(idx0, semi0)
        rwait(r20, semg0)
        pltpu.sync_copy(r20, qd_sh.at[idx0.at[1]], add=True)
        iwait(idx1, semi1)
        rwait(r21, semg1)
        pltpu.sync_copy(r21, qd_sh.at[idx1.at[1]], add=True)
        if with_tail:
            pltpu.sync_copy(ei_hbm.at[:, pl.ds(tbase + NB * B, BT)], idx_t)
            pltpu.sync_copy(r2_hbm.at[pl.ds(tbase + NB * B, BT)], r2_t)
            pltpu.sync_copy(r2_t, qd_sh.at[idx_t.at[1]], add=True)

    @pl.when(c == 0)
    def _():
        p2_pipeline(g2a_hbm)
        qd_pipeline(0, False)

    @pl.when(c == 1)
    def _():
        p2_pipeline(g2b_hbm)
        qd_pipeline(QD_NB, True)

    plsc.subcore_barrier()

    pltpu.sync_copy(p2_sh.at[pl.ds(base_r, ROWS_PER_TILE)],
                    p2_out.at[c, pl.ds(base_r, ROWS_PER_TILE)])
    pltpu.sync_copy(qd_sh.at[pl.ds(base_r, ROWS_PER_TILE)],
                    qd_out.at[c, pl.ds(base_r, ROWS_PER_TILE)])


def _post_body(h_ref, g2a_ref, g2b_ref, p2p_ref, qdp_ref, waht_ref, wa1t_ref,
               wa2t_ref, ba_ref, out_ref):
    p2 = jnp.concatenate([p2p_ref[0], p2p_ref[1]], axis=1)
    qd = qdp_ref[0] + qdp_ref[1]
    deg = qd[:, DR:DR + 1]
    qr = qd[:, 0:DR]
    g2 = jnp.concatenate([g2a_ref[...], g2b_ref[...]], axis=1)
    hs1 = p2 + deg * g2
    acc = jnp.dot(h_ref[...], waht_ref[...], preferred_element_type=jnp.float32)
    acc = acc + jnp.dot(hs1, wa1t_ref[...], preferred_element_type=jnp.float32)
    acc = acc + jnp.dot(qr, wa2t_ref[...], preferred_element_type=jnp.float32)
    out_ref[...] = jnp.maximum(acc + ba_ref[...], 0.0)


def _post(h, g2a, g2b, p2p, qdp, wah_t, wa1_t, wa2_t, b_apply):
    grid = (N // _NODE_BLK,)
    return pl.pallas_call(
        _post_body,
        grid=grid,
        in_specs=[
            pl.BlockSpec((_NODE_BLK, D), lambda i: (i, 0)),
            pl.BlockSpec((_NODE_BLK, DH), lambda i: (i, 0)),
            pl.BlockSpec((_NODE_BLK, DH), lambda i: (i, 0)),
            pl.BlockSpec((NC, _NODE_BLK, DH), lambda i: (0, i, 0)),
            pl.BlockSpec((NC, _NODE_BLK, 2 * DR), lambda i: (0, i, 0)),
            pl.BlockSpec((D, D), lambda i: (0, 0)),
            pl.BlockSpec((D, D), lambda i: (0, 0)),
            pl.BlockSpec((DR, D), lambda i: (0, 0)),
            pl.BlockSpec((1, D), lambda i: (0, 0)),
        ],
        out_specs=pl.BlockSpec((_NODE_BLK, D), lambda i: (i, 0)),
        out_shape=jax.ShapeDtypeStruct((N, D), jnp.float32),
    )(h, g2a, g2b, p2p, qdp, wah_t, wa1_t, wa2_t, b_apply)


def kernel(x, att_node, edge_index, rel, att_edge, W_node, b_node, W_rel,
           b_rel, W_apply, b_apply):
    ei32 = edge_index.astype(jnp.int32)
    h, g2a, g2b = _node_pre(x, att_node, W_node.T, b_node.reshape(1, D))
    r2aug = _edge_pre(rel, att_edge, W_rel.T, b_rel.reshape(1, DR))
    p2p, qdp = _sc_edge_agg(g2a, g2b, ei32, r2aug)
    return _post(h, g2a, g2b, p2p, qdp,
                 W_apply[:, :D].T, W_apply[:, D:2 * D].T, W_apply[:, 2 * D:].T,
                 b_apply.reshape(1, D))
